# baseline (speedup 1.0000x reference)
import functools

import jax
import jax.numpy as jnp
from jax import lax
from jax.experimental import pallas as pl
from jax.experimental.pallas import tpu as pltpu
from jax.experimental.pallas import tpu_sc as plsc

_NC = 2
_NS = 16
_NW = _NC * _NS
_CHUNK = 128


def _gather_body(b_per_w, n_chunks, W,
                 uidx_hbm, iidx_hbm, uemb_hbm, iemb_hbm,
                 uout_hbm, iout_hbm,
                 uidx_v, iidx_v, urows_v, irows_v, sem0, sem1):
    sems = (sem0, sem1)
    wid = lax.axis_index("s") * _NC + lax.axis_index("c")
    base = wid * b_per_w
    pltpu.sync_copy(uidx_hbm.at[wid], uidx_v)
    pltpu.sync_copy(iidx_hbm.at[wid], iidx_v)
    copies = {}
    for c in range(n_chunks):
        slot = c % 2
        if c >= 2:
            for cp in copies.pop(c - 2):
                cp.wait()
            dst = pl.ds(base + (c - 2) * _CHUNK, _CHUNK)
            pltpu.sync_copy(urows_v.at[slot], uout_hbm.at[dst])
            pltpu.sync_copy(irows_v.at[slot], iout_hbm.at[dst])
        pair = (
            pltpu.make_async_copy(
                uemb_hbm.at[uidx_v.at[c]], urows_v.at[slot], sems[slot]),
            pltpu.make_async_copy(
                iemb_hbm.at[iidx_v.at[c]], irows_v.at[slot], sems[slot]),
        )
        for cp in pair:
            cp.start()
        copies[c] = pair
    for c in sorted(copies):
        for cp in copies[c]:
            cp.wait()
        dst = pl.ds(base + c * _CHUNK, _CHUNK)
        pltpu.sync_copy(urows_v.at[c % 2], uout_hbm.at[dst])
        pltpu.sync_copy(irows_v.at[c % 2], iout_hbm.at[dst])


@functools.partial(jax.jit, static_argnums=(4, 5))
def _sc_gather(uidx, iidx, uemb, iemb, B, W):
    b_per_w = B // _NW
    n_chunks = b_per_w // _CHUNK
    mesh = plsc.VectorSubcoreMesh(core_axis_name="c", subcore_axis_name="s")
    body = functools.partial(_gather_body, b_per_w, n_chunks, W)
    kern = pl.kernel(
        body,
        out_type=[
            jax.ShapeDtypeStruct((B, W), jnp.float32),
            jax.ShapeDtypeStruct((B, W), jnp.float32),
        ],
        mesh=mesh,
        scratch_types=[
            pltpu.VMEM((n_chunks, _CHUNK), jnp.int32),
            pltpu.VMEM((n_chunks, _CHUNK), jnp.int32),
            pltpu.VMEM((2, _CHUNK, W), jnp.float32),
            pltpu.VMEM((2, _CHUNK, W), jnp.float32),
            pltpu.SemaphoreType.DMA,
            pltpu.SemaphoreType.DMA,
        ],
    )
    uidx3 = uidx.reshape(_NW, n_chunks, _CHUNK)
    iidx3 = iidx.reshape(_NW, n_chunks, _CHUNK)
    return kern(uidx3, iidx3, uemb, iemb)


def _mlp_body(u_ref, i_ref, ur_ref, ir_ref, w1a_ref, w1b_ref, b1_ref,
              w2_ref, b2_ref, wp_ref, bp_ref, o_ref):
    D = 32
    ur = ur_ref[...]
    ir = ir_ref[...]
    u = jnp.zeros(u_ref.shape[:1] + (D,), jnp.float32)
    it = jnp.zeros_like(u)
    for k in range(4):
        u = u + jnp.where(ur == k, u_ref[:, k * D:(k + 1) * D], 0.0)
        it = it + jnp.where(ir == k, i_ref[:, k * D:(k + 1) * D], 0.0)
    h1 = jnp.dot(u, w1a_ref[...], preferred_element_type=jnp.float32)
    h1 += jnp.dot(it, w1b_ref[...], preferred_element_type=jnp.float32)
    h1 = jnp.maximum(h1 + b1_ref[...], 0.0)
    h2 = jnp.dot(h1, w2_ref[...], preferred_element_type=jnp.float32)
    h2 = jnp.maximum(h2 + b2_ref[...], 0.0)
    p = jnp.dot(h2, wp_ref[...], preferred_element_type=jnp.float32)
    o_ref[...] = jax.nn.sigmoid(p + bp_ref[...])


def _tc_mlp(u, it, urem, irem, W1, b1, W2, b2, Wp, bp, B, BK):
    D = 32
    w1a = W1[:, :D].T
    w1b = W1[:, D:].T
    w2 = W2.T
    wp = Wp.T
    b1r = b1.reshape(1, -1)
    b2r = b2.reshape(1, -1)
    bpr = bp.reshape(1, -1)
    grid = B // BK

    def full(shape):
        return pl.BlockSpec(shape, lambda i: (0,) * len(shape))

    out = pl.pallas_call(
        _mlp_body,
        grid=(grid,),
        in_specs=[
            pl.BlockSpec((BK, 128), lambda i: (i, 0)),
            pl.BlockSpec((BK, 128), lambda i: (i, 0)),
            pl.BlockSpec((BK, 1), lambda i: (i, 0)),
            pl.BlockSpec((BK, 1), lambda i: (i, 0)),
            full(w1a.shape), full(w1b.shape), full(b1r.shape),
            full(w2.shape), full(b2r.shape),
            full(wp.shape), full(bpr.shape),
        ],
        out_specs=pl.BlockSpec((BK, 1), lambda i: (i, 0)),
        out_shape=jax.ShapeDtypeStruct((B, 1), jnp.float32),
    )(u, it, urem, irem, w1a, w1b, b1r, w2, b2r, wp, bpr)
    return out


def kernel(user_indices, item_indices, user_emb, item_emb,
           W1, b1, W2, b2, Wp, bp):
    B = user_indices.shape[0]
    V, D = user_emb.shape
    uidx = user_indices.astype(jnp.int32)
    iidx = item_indices.astype(jnp.int32)
    rows_per_phys = 128 // D
    uemb128 = user_emb.reshape(V // rows_per_phys, 128)
    iemb128 = item_emb.reshape(V // rows_per_phys, 128)
    u_rows, i_rows = _sc_gather(
        uidx // rows_per_phys, iidx // rows_per_phys,
        uemb128, iemb128, B, 128)
    urem = (uidx % rows_per_phys).reshape(B, 1)
    irem = (iidx % rows_per_phys).reshape(B, 1)
    out = _tc_mlp(u_rows, i_rows, urem, irem,
                  W1, b1, W2, b2, Wp, bp, B, 2048)
    return jnp.squeeze(out, axis=-1)


# speedup vs baseline: 1.4914x; 1.4914x over previous
import functools

import jax
import jax.numpy as jnp
from jax import lax
from jax.experimental import pallas as pl
from jax.experimental.pallas import tpu as pltpu
from jax.experimental.pallas import tpu_sc as plsc

_NC = 2
_NS = 16
_NW = _NC * _NS
_K = 16


def _fire_chunk(emb_refs, idx_refs, rows, sem, chunk, k):
    copies = []
    for tbl in range(2):
        iv = idx_refs[tbl][pl.ds(chunk * _K, _K)]
        for j in range(_K):
            cp = pltpu.make_async_copy(
                emb_refs[tbl].at[iv[j]], rows[tbl].at[k, j], sem)
            cp.start()
            copies.append(cp)
    return copies


def _drain_chunk(copies, rows, out_refs, base, chunk, k):
    for cp in copies:
        cp.wait()
    dst = pl.ds(base + chunk * _K, _K)
    pltpu.sync_copy(rows[0].at[k], out_refs[0].at[dst])
    pltpu.sync_copy(rows[1].at[k], out_refs[1].at[dst])


def _gather_body(b_per_w,
                 uidx_hbm, iidx_hbm, uemb_hbm, iemb_hbm,
                 uout_hbm, iout_hbm,
                 uidx_v, iidx_v, urows_v, irows_v, sem0, sem1):
    wid = lax.axis_index("s") * _NC + lax.axis_index("c")
    base = wid * b_per_w
    pltpu.sync_copy(uidx_hbm.at[wid], uidx_v)
    pltpu.sync_copy(iidx_hbm.at[wid], iidx_v)
    n_chunks = b_per_w // _K
    embs = (uemb_hbm, iemb_hbm)
    idxs = (uidx_v, iidx_v)
    rows = (urows_v, irows_v)
    outs = (uout_hbm, iout_hbm)
    sems = (sem0, sem1)
    pending0 = _fire_chunk(embs, idxs, rows, sems[0], 0, 0)
    for c in range(n_chunks // 2):
        pending1 = _fire_chunk(embs, idxs, rows, sems[1], 2 * c + 1, 1)
        _drain_chunk(pending0, rows, outs, base, 2 * c, 0)
        if 2 * c + 2 < n_chunks:
            pending0 = _fire_chunk(embs, idxs, rows, sems[0], 2 * c + 2, 0)
        _drain_chunk(pending1, rows, outs, base, 2 * c + 1, 1)


@functools.partial(jax.jit, static_argnums=(4,))
def _sc_gather(uidx, iidx, uemb, iemb, B):
    b_per_w = B // _NW
    mesh = plsc.VectorSubcoreMesh(core_axis_name="c", subcore_axis_name="s")
    body = functools.partial(_gather_body, b_per_w)
    kern = pl.kernel(
        body,
        out_type=[
            jax.ShapeDtypeStruct((B, 32), jnp.float32),
            jax.ShapeDtypeStruct((B, 32), jnp.float32),
        ],
        mesh=mesh,
        scratch_types=[
            pltpu.VMEM((b_per_w,), jnp.int32),
            pltpu.VMEM((b_per_w,), jnp.int32),
            pltpu.VMEM((2, _K, 32), jnp.float32),
            pltpu.VMEM((2, _K, 32), jnp.float32),
            pltpu.SemaphoreType.DMA,
            pltpu.SemaphoreType.DMA,
        ],
        compiler_params=pltpu.CompilerParams(needs_layout_passes=False),
    )
    return kern(uidx.reshape(_NW, b_per_w), iidx.reshape(_NW, b_per_w),
                uemb, iemb)


def _mlp_body(u_ref, i_ref, w1a_ref, w1b_ref, b1_ref,
              w2_ref, b2_ref, wp_ref, bp_ref, o_ref):
    u = u_ref[...]
    it = i_ref[...]
    h1 = jnp.dot(u, w1a_ref[...], preferred_element_type=jnp.float32)
    h1 += jnp.dot(it, w1b_ref[...], preferred_element_type=jnp.float32)
    h1 = jnp.maximum(h1 + b1_ref[...], 0.0)
    h2 = jnp.dot(h1, w2_ref[...], preferred_element_type=jnp.float32)
    h2 = jnp.maximum(h2 + b2_ref[...], 0.0)
    p = jnp.dot(h2, wp_ref[...], preferred_element_type=jnp.float32)
    o_ref[...] = jax.nn.sigmoid(p + bp_ref[...])


def _tc_mlp(u, it, W1, b1, W2, b2, Wp, bp, B, BK):
    D = 32
    w1a = W1[:, :D].T
    w1b = W1[:, D:].T
    w2 = W2.T
    wp = Wp.T
    b1r = b1.reshape(1, -1)
    b2r = b2.reshape(1, -1)
    bpr = bp.reshape(1, -1)
    grid = B // BK

    def full(shape):
        return pl.BlockSpec(shape, lambda i: (0,) * len(shape))

    out = pl.pallas_call(
        _mlp_body,
        grid=(grid,),
        in_specs=[
            pl.BlockSpec((BK, D), lambda i: (i, 0)),
            pl.BlockSpec((BK, D), lambda i: (i, 0)),
            full(w1a.shape), full(w1b.shape), full(b1r.shape),
            full(w2.shape), full(b2r.shape),
            full(wp.shape), full(bpr.shape),
        ],
        out_specs=pl.BlockSpec((BK, 1), lambda i: (i, 0)),
        out_shape=jax.ShapeDtypeStruct((B, 1), jnp.float32),
    )(u, it, w1a, w1b, b1r, w2, b2r, wp, bpr)
    return out


def kernel(user_indices, item_indices, user_emb, item_emb,
           W1, b1, W2, b2, Wp, bp):
    B = user_indices.shape[0]
    uidx = user_indices.astype(jnp.int32)
    iidx = item_indices.astype(jnp.int32)
    u_rows, i_rows = _sc_gather(uidx, iidx, user_emb, item_emb, B)
    out = _tc_mlp(u_rows, i_rows, W1, b1, W2, b2, Wp, bp, B, 2048)
    return jnp.squeeze(out, axis=-1)
